# baseline (speedup 1.0000x reference)
import functools

import jax
import jax.numpy as jnp
from jax import lax
from jax.experimental import pallas as pl
from jax.experimental.pallas import tpu as pltpu
from jax.experimental.pallas import tpu_sc as plsc

N_NODES = 10000
N_EDGES = 320000
D = 128
L = 16
NG = D // L
NC_E = 4
EPT = N_EDGES // NC_E
CE = 128
NCHUNKS = EPT // CE
HALF = N_NODES // 2
TRASH = HALF


def _tc_pool(x, W_pool, b_pool):
    BM = 1000

    def body(x_ref, w_ref, b_ref, o_ref):
        o_ref[...] = jnp.maximum(
            jnp.dot(x_ref[...], w_ref[...], preferred_element_type=jnp.float32)
            + b_ref[...], 0.0)

    return pl.pallas_call(
        body,
        grid=(N_NODES // BM,),
        in_specs=[
            pl.BlockSpec((BM, D), lambda i: (i, 0)),
            pl.BlockSpec((D, D), lambda i: (0, 0)),
            pl.BlockSpec((1, D), lambda i: (0, 0)),
        ],
        out_specs=pl.BlockSpec((BM, D), lambda i: (i, 0)),
        out_shape=jax.ShapeDtypeStruct((N_NODES, D), jnp.float32),
    )(x, W_pool, b_pool.reshape(1, D))


def _tc_out(x, partials, W_self, W_neigh, b):
    BM = 1000

    def body(x_ref, p_ref, ws_ref, wn_ref, b_ref, o_ref):
        agg = jnp.max(p_ref[...], axis=0)
        agg = jnp.where(jnp.isfinite(agg), agg, 0.0)
        o_ref[...] = (
            jnp.dot(x_ref[...], ws_ref[...], preferred_element_type=jnp.float32)
            + jnp.dot(agg, wn_ref[...], preferred_element_type=jnp.float32)
            + b_ref[...])

    return pl.pallas_call(
        body,
        grid=(N_NODES // BM,),
        in_specs=[
            pl.BlockSpec((BM, D), lambda i: (i, 0)),
            pl.BlockSpec((NC_E, BM, D), lambda i: (0, i, 0)),
            pl.BlockSpec((D, D), lambda i: (0, 0)),
            pl.BlockSpec((D, D), lambda i: (0, 0)),
            pl.BlockSpec((1, D), lambda i: (0, 0)),
        ],
        out_specs=pl.BlockSpec((BM, D), lambda i: (i, 0)),
        out_shape=jax.ShapeDtypeStruct((N_NODES, D), jnp.float32),
    )(x, partials, W_self, W_neigh, b.reshape(1, D))


def _sc_segmax(h2, src, dst):
    mesh = plsc.VectorSubcoreMesh(core_axis_name="c", subcore_axis_name="s")

    @functools.partial(
        pl.kernel,
        mesh=mesh,
        compiler_params=pltpu.CompilerParams(use_tc_tiling_on_sc=False),
        out_type=jax.ShapeDtypeStruct((NC_E, N_NODES, NG, L), jnp.float32),
        scratch_types=[
            pltpu.VMEM((HALF + 1, L), jnp.float32),
            pltpu.VMEM((CE, L), jnp.float32),
            pltpu.VMEM((CE,), jnp.int32),
            pltpu.VMEM((CE,), jnp.int32),
            pltpu.VMEM((CE,), jnp.int32),
            pltpu.SemaphoreType.DMA,
        ],
    )
    def k(h2_hbm, src_hbm, dst_hbm, out_hbm, agg_v, msg_v, idx_v, srcb_v,
          dstb_v, sem):
        wid = lax.axis_index("s") * 2 + lax.axis_index("c")
        g = wid % NG
        c = wid // NG
        ebase = c * EPT
        neg = jnp.full((L,), -jnp.inf, jnp.float32)

        for p in range(2):
            nbase = p * HALF

            def init_body(i, carry):
                agg_v[i] = neg
                return carry
            lax.fori_loop(0, HALF + 1, init_body, 0)

            def chunk_body(j, carry):
                e0 = ebase + j * CE
                pltpu.sync_copy(src_hbm.at[pl.ds(e0, CE)], srcb_v)
                pltpu.sync_copy(dst_hbm.at[pl.ds(e0, CE)], dstb_v)
                for q in range(CE // L):
                    idx_v[pl.ds(q * L, L)] = srcb_v[pl.ds(q * L, L)] * NG + g
                pltpu.async_copy(h2_hbm.at[idx_v], msg_v, sem).wait()

                def rmw_body(grp, carry2):
                    d16 = dstb_v[pl.ds(grp * L, L)]
                    for kk in range(L):
                        a = d16[kk] - nbase
                        row = jnp.where((a >= 0) & (a < HALF), a, TRASH)
                        m = msg_v[grp * L + kk]
                        agg_v[row] = jnp.maximum(agg_v[row], m)
                    return carry2
                lax.fori_loop(0, CE // L, rmw_body, 0)
                return carry
            lax.fori_loop(0, NCHUNKS, chunk_body, 0)

            pltpu.sync_copy(agg_v.at[pl.ds(0, HALF)],
                            out_hbm.at[c, pl.ds(nbase, HALF), g])

    return k(h2, src, dst)


def kernel(x, edge_index, W_pool, b_pool, W_self, W_neigh, b):
    h = _tc_pool(x, W_pool, b_pool)
    h2 = h.reshape(N_NODES * NG, L)
    partials = _sc_segmax(h2, edge_index[0], edge_index[1])
    return _tc_out(x, partials.reshape(NC_E, N_NODES, D), W_self, W_neigh, b)


# speedup vs baseline: 2.8135x; 2.8135x over previous
import functools

import jax
import jax.numpy as jnp
from jax import lax
from jax.experimental import pallas as pl
from jax.experimental.pallas import tpu as pltpu
from jax.experimental.pallas import tpu_sc as plsc

N_NODES = 10000
N_EDGES = 320000
D = 128
L = 16
NG = D // L
NC_E = 4
EPT = N_EDGES // NC_E
CE = 160
GH = CE // 2
G16 = CE // L
NCHUNKS = EPT // CE
NBLK = NCHUNKS // 4
HALF = N_NODES // 2
TRASH = HALF


def _tc_pool(x, W_pool, b_pool):
    BM = 1000

    def body(x_ref, w_ref, b_ref, o_ref):
        o_ref[...] = jnp.maximum(
            jnp.dot(x_ref[...], w_ref[...], preferred_element_type=jnp.float32)
            + b_ref[...], 0.0)

    return pl.pallas_call(
        body,
        grid=(N_NODES // BM,),
        in_specs=[
            pl.BlockSpec((BM, D), lambda i: (i, 0)),
            pl.BlockSpec((D, D), lambda i: (0, 0)),
            pl.BlockSpec((1, D), lambda i: (0, 0)),
        ],
        out_specs=pl.BlockSpec((BM, D), lambda i: (i, 0)),
        out_shape=jax.ShapeDtypeStruct((N_NODES, D), jnp.float32),
    )(x, W_pool, b_pool.reshape(1, D))


def _tc_out(x, partials, W_self, W_neigh, b):
    BM = 1000

    def body(x_ref, p_ref, ws_ref, wn_ref, b_ref, o_ref):
        agg = jnp.max(p_ref[...], axis=0)
        agg = jnp.where(jnp.isfinite(agg), agg, 0.0)
        o_ref[...] = (
            jnp.dot(x_ref[...], ws_ref[...], preferred_element_type=jnp.float32)
            + jnp.dot(agg, wn_ref[...], preferred_element_type=jnp.float32)
            + b_ref[...])

    return pl.pallas_call(
        body,
        grid=(N_NODES // BM,),
        in_specs=[
            pl.BlockSpec((BM, D), lambda i: (i, 0)),
            pl.BlockSpec((NC_E, BM, D), lambda i: (0, i, 0)),
            pl.BlockSpec((D, D), lambda i: (0, 0)),
            pl.BlockSpec((D, D), lambda i: (0, 0)),
            pl.BlockSpec((1, D), lambda i: (0, 0)),
        ],
        out_specs=pl.BlockSpec((BM, D), lambda i: (i, 0)),
        out_shape=jax.ShapeDtypeStruct((N_NODES, D), jnp.float32),
    )(x, partials, W_self, W_neigh, b.reshape(1, D))


def _sc_segmax(h2, src, dst):
    mesh = plsc.VectorSubcoreMesh(core_axis_name="c", subcore_axis_name="s")

    @functools.partial(
        pl.kernel,
        mesh=mesh,
        compiler_params=pltpu.CompilerParams(use_tc_tiling_on_sc=False),
        out_type=jax.ShapeDtypeStruct((NC_E, N_NODES, NG, L), jnp.float32),
        scratch_types=[
            pltpu.VMEM((HALF + 1, L), jnp.float32),
            pltpu.VMEM((2, CE, L), jnp.float32),
            pltpu.VMEM((2, 2, GH), jnp.int32),
            pltpu.VMEM((4, CE), jnp.int32),
            pltpu.VMEM((4, CE), jnp.int32),
            pltpu.SemaphoreType.DMA,
            pltpu.SemaphoreType.DMA,
            pltpu.SemaphoreType.DMA,
            pltpu.SemaphoreType.DMA,
            pltpu.SemaphoreType.DMA,
            pltpu.SemaphoreType.DMA,
        ],
    )
    def k(h2_hbm, src_hbm, dst_hbm, out_hbm, agg_v, msg_v, idx_v, srcb_v,
          dstb_v, sd0, sd1, sd2, sd3, sg0, sg1):
        s_sd = [sd0, sd1, sd2, sd3]
        s_g = [sg0, sg1]
        wid = lax.axis_index("s") * 2 + lax.axis_index("c")
        g = wid % NG
        c = wid // NG
        ebase = c * EPT
        neg = jnp.full((L,), -jnp.inf, jnp.float32)

        def fire_sd(slot, j):
            e0 = ebase + j * CE
            pltpu.async_copy(src_hbm.at[pl.ds(e0, CE)], srcb_v.at[slot],
                             s_sd[slot])
            pltpu.async_copy(dst_hbm.at[pl.ds(e0, CE)], dstb_v.at[slot],
                             s_sd[slot])

        def wait_sd(slot):
            pltpu.make_async_copy(src_hbm.at[pl.ds(0, CE)], srcb_v.at[slot],
                                  s_sd[slot]).wait()
            pltpu.make_async_copy(dst_hbm.at[pl.ds(0, CE)], dstb_v.at[slot],
                                  s_sd[slot]).wait()

        def fire_gather(gs, slot):
            for q in range(G16):
                half, off = divmod(q * L, GH)
                idx_v[gs, half, pl.ds(off, L)] = (
                    srcb_v[slot, pl.ds(q * L, L)] * NG + g)
            for half in range(2):
                pltpu.async_copy(h2_hbm.at[idx_v.at[gs, half]],
                                 msg_v.at[gs, pl.ds(half * GH, GH)], s_g[gs])

        def wait_gather(gs):
            for half in range(2):
                pltpu.make_async_copy(h2_hbm.at[idx_v.at[gs, half]],
                                      msg_v.at[gs, pl.ds(half * GH, GH)],
                                      s_g[gs]).wait()

        def rmw(gs, slot, nbase):
            def rmw_body(grp, carry2):
                d16 = dstb_v[slot, pl.ds(grp * L, L)]
                for kk in range(L):
                    a = d16[kk] - nbase
                    row = jnp.where((a >= 0) & (a < HALF), a, TRASH)
                    m = msg_v[gs, grp * L + kk]
                    agg_v[row] = jnp.maximum(agg_v[row], m)
                return carry2
            lax.fori_loop(0, G16, rmw_body, 0)

        for p in range(2):
            nbase = p * HALF

            def init_body(i, carry):
                agg_v[i] = neg
                return carry
            lax.fori_loop(0, HALF + 1, init_body, 0)

            fire_sd(0, 0)
            fire_sd(1, 1)
            wait_sd(0)
            fire_gather(0, 0)

            def blk(jj, carry):
                for u in range(4):
                    j = jj * 4 + u
                    @pl.when(j + 2 < NCHUNKS)
                    def _():
                        fire_sd((u + 2) % 4, j + 2)

                    @pl.when(j + 1 < NCHUNKS)
                    def _():
                        wait_sd((u + 1) % 4)
                        fire_gather((u + 1) % 2, (u + 1) % 4)

                    wait_gather(u % 2)
                    rmw(u % 2, u % 4, nbase)
                return carry
            lax.fori_loop(0, NBLK, blk, 0)

            pltpu.sync_copy(agg_v.at[pl.ds(0, HALF)],
                            out_hbm.at[c, pl.ds(nbase, HALF), g])

    return k(h2, src, dst)


def kernel(x, edge_index, W_pool, b_pool, W_self, W_neigh, b):
    h = _tc_pool(x, W_pool, b_pool)
    h2 = h.reshape(N_NODES * NG, L)
    partials = _sc_segmax(h2, edge_index[0], edge_index[1])
    return _tc_out(x, partials.reshape(NC_E, N_NODES, D), W_self, W_neigh, b)
